# baseline (speedup 1.0000x reference)
import functools

import jax
import jax.numpy as jnp
from jax import lax
from jax.experimental import pallas as pl
from jax.experimental.pallas import tpu as pltpu
from jax.experimental.pallas import tpu_sc as plsc

N = 10000
D = 128
DE = 16

NCORES = 2
NSUB = 16
NW = NCORES * NSUB
CH = 128
NB = 4
NBS = 2

N_PAD = 10240
E_PAD = 327680
S = 5
E_S = E_PAD // S
RPW = E_S // NW
NCH = RPW // CH
NGROUP = NCH // NB
RPS = N_PAD // NSUB
BE = 2048
BN = 2048

_SC_PARAMS = pltpu.CompilerParams(needs_layout_passes=False)


def _gelu(x):
    return jax.nn.gelu(x)


def _proj_body(x_ref, w_ref, o_ref):
    o_ref[...] = jnp.dot(x_ref[...], w_ref[...], preferred_element_type=jnp.float32)


def _project(nodes_pad, w):
    return pl.pallas_call(
        _proj_body,
        out_shape=jax.ShapeDtypeStruct((N_PAD, D), jnp.float32),
    )(nodes_pad, w)


def _sc_gather(table, idx2d, dst2d):
    mesh = plsc.VectorSubcoreMesh(core_axis_name="c", subcore_axis_name="s")

    @functools.partial(
        pl.kernel,
        out_type=(
            jax.ShapeDtypeStruct((E_S, D), jnp.float32),
            jax.ShapeDtypeStruct((NW, N_PAD), jnp.float32),
        ),
        mesh=mesh,
        compiler_params=_SC_PARAMS,
        scratch_types=[
            pltpu.VMEM((NCH, CH), jnp.int32),
            pltpu.VMEM((NCH, CH), jnp.int32),
            pltpu.VMEM((N_PAD,), jnp.float32),
        ]
        + [pltpu.VMEM((CH, D), jnp.float32) for _ in range(NB)]
        + [pltpu.SemaphoreType.DMA for _ in range(2 * NB)],
    )
    def k(table_hbm, idx_hbm, dst_hbm, out_hbm, counts_hbm, idx_all, dst_all, cnt_v, *rest):
        bufs = rest[:NB]
        gs = rest[NB:2 * NB]
        ws = rest[2 * NB:]
        wid = lax.axis_index("s") * NCORES + lax.axis_index("c")
        base = wid * RPW
        pltpu.sync_copy(idx_hbm.at[pl.ds(wid * NCH, NCH)], idx_all)
        pltpu.sync_copy(dst_hbm.at[pl.ds(wid * NCH, NCH)], dst_all)
        woff = wid * (N_PAD // NW)
        pltpu.sync_copy(table_hbm.at[pl.ds(woff, CH), :], bufs[0])
        pltpu.sync_copy(table_hbm.at[pl.ds(woff + CH, CH), :], bufs[1])
        pltpu.sync_copy(table_hbm.at[pl.ds(woff + 2 * CH, 64), :], bufs[2].at[pl.ds(0, 64), :])
        z16 = jnp.zeros((16,), jnp.float32)
        ones16 = jnp.ones((16,), jnp.float32)

        def zcnt(i, carry):
            cnt_v[pl.ds(i * 16, 16)] = z16
            return carry

        lax.fori_loop(0, N_PAD // 16, zcnt, 0)

        def count(j, carry):
            for kk in range(CH // 16):
                iv = dst_all[j, pl.ds(kk * 16, 16)]
                plsc.addupdate_scatter(cnt_v, [iv], ones16)
            return carry

        def g_start(j, b):
            pltpu.make_async_copy(table_hbm.at[idx_all.at[j]], bufs[b], gs[b]).start()

        def g_wait(j, b):
            pltpu.make_async_copy(table_hbm.at[idx_all.at[j]], bufs[b], gs[b]).wait()

        def w_start(j, b):
            pltpu.make_async_copy(
                bufs[b], out_hbm.at[pl.ds(base + j * CH, CH), :], ws[b]
            ).start()

        def w_wait(j, b):
            pltpu.make_async_copy(
                bufs[b], out_hbm.at[pl.ds(base + j * CH, CH), :], ws[b]
            ).wait()

        LA = NB - 2

        def step(j, b, start_next, drain_prev):
            bn = (b + LA) % NB
            if start_next:
                if drain_prev:
                    w_wait(j + LA - NB, bn)
                g_start(j + LA, bn)
            g_wait(j, b)
            w_start(j, b)

        for j in range(LA):
            g_start(j, j)
        for j in range(NB):
            step(j, j, True, j + LA >= NB)

        def body(g, carry):
            for b in range(NB):
                step(g * NB + b, b, True, True)
            return carry

        lax.fori_loop(1, NGROUP - 1, body, 0)

        for b in range(NB):
            j = (NGROUP - 1) * NB + b
            step(j, b, j + LA < NCH, True)

        lax.fori_loop(0, NCH, count, 0)
        pltpu.sync_copy(cnt_v, counts_hbm.at[wid])
        for b in range(NB):
            w_wait(NCH - NB + b, (NCH - NB + b) % NB)

    return k(table, idx2d, dst2d)


def _edge_ffn_body(g_ref, ef_ref, w0n_ref, w0e_ref, b0_ref, w1_ref, b1_ref, o_ref):
    q = lax.dot_general(
        ef_ref[...], w0e_ref[...], (((0,), (0,)), ((), ())),
        preferred_element_type=jnp.float32,
    )
    pre = (
        jnp.dot(g_ref[...], w0n_ref[...], preferred_element_type=jnp.float32)
        + q
        + b0_ref[...]
    )
    h = _gelu(pre)
    o_ref[...] = _gelu(
        jnp.dot(h, w1_ref[...], preferred_element_type=jnp.float32) + b1_ref[...]
    )


def _edge_ffn(g, efT, ef_col0, w0n, w0e, b0, w1, b1):
    nblk = E_S // BE
    blk0 = ef_col0 // BE
    return pl.pallas_call(
        _edge_ffn_body,
        grid=(nblk,),
        in_specs=[
            pl.BlockSpec((BE, D), lambda i: (i, 0)),
            pl.BlockSpec((DE, BE), lambda i: (0, blk0 + i)),
            pl.BlockSpec((D, D), lambda i: (0, 0)),
            pl.BlockSpec((DE, D), lambda i: (0, 0)),
            pl.BlockSpec((1, D), lambda i: (0, 0)),
            pl.BlockSpec((D, D), lambda i: (0, 0)),
            pl.BlockSpec((1, D), lambda i: (0, 0)),
        ],
        out_specs=pl.BlockSpec((BE, D), lambda i: (i, 0)),
        out_shape=jax.ShapeDtypeStruct((E_S, D), jnp.float32),
    )(g, efT, w0n, w0e, b0, w1, b1)


def _sc_scatter(msgs, dst2d):
    mesh = plsc.VectorSubcoreMesh(core_axis_name="c", subcore_axis_name="s")
    ngroup = NCH // NBS

    @functools.partial(
        pl.kernel,
        out_type=jax.ShapeDtypeStruct((NCORES, N_PAD, D), jnp.float32),
        mesh=mesh,
        compiler_params=_SC_PARAMS,
        scratch_types=[pltpu.VMEM((NCH, CH), jnp.int32)]
        + [pltpu.VMEM((CH, D), jnp.float32) for _ in range(NBS)]
        + [pltpu.VMEM_SHARED((N_PAD, D), jnp.float32)]
        + [pltpu.SemaphoreType.DMA for _ in range(NBS)],
    )
    def k(msgs_hbm, dst_hbm, sums_hbm, idx_all, *rest):
        ms = rest[:NBS]
        acc_sh = rest[NBS]
        ss = rest[NBS + 1:]
        cid = lax.axis_index("c")
        sid = lax.axis_index("s")
        wid = sid * NCORES + cid
        z16 = jnp.zeros((16,), jnp.float32)

        def zbuf(i, carry):
            r = i // (D // 16)
            c = (i % (D // 16)) * 16
            ms[0][r, pl.ds(c, 16)] = z16
            return carry

        lax.fori_loop(0, CH * (D // 16), zbuf, 0)

        for j in range(RPS // CH):
            pltpu.sync_copy(ms[0], acc_sh.at[pl.ds(sid * RPS + j * CH, CH), :])
        plsc.subcore_barrier()

        pltpu.sync_copy(dst_hbm.at[pl.ds(wid * NCH, NCH)], idx_all)
        base = wid * RPW

        def l_start(j, b):
            pltpu.make_async_copy(
                msgs_hbm.at[pl.ds(base + j * CH, CH), :], ms[b], ss[b]
            ).start()

        def l_wait(j, b):
            pltpu.make_async_copy(
                msgs_hbm.at[pl.ds(base + j * CH, CH), :], ms[b], ss[b]
            ).wait()

        def consume(j, b):
            pltpu.sync_copy(ms[b], acc_sh.at[idx_all.at[j]], add=True)

        for b in range(NBS):
            l_start(b, b)

        def group(g, carry):
            for b in range(NBS):
                j = g * NBS + b
                l_wait(j, b)
                consume(j, b)
                l_start((g + 1) * NBS + b, b)
            return carry

        lax.fori_loop(0, ngroup - 1, group, 0)

        gl = ngroup - 1
        for b in range(NBS):
            j = gl * NBS + b
            l_wait(j, b)
            consume(j, b)
        plsc.subcore_barrier()

        for j in range(RPS // CH):
            r0 = sid * RPS + j * CH
            pltpu.sync_copy(acc_sh.at[pl.ds(r0, CH), :], ms[0])
            pltpu.sync_copy(ms[0], sums_hbm.at[cid, pl.ds(r0, CH), :])

    return k(msgs, dst2d)


def _node_ffn_body(*refs):
    x_ref = refs[0]
    s_refs = refs[1:1 + S]
    c_refs = refs[1 + S:1 + 2 * S]
    w0a_ref, w0b_ref, b0_ref, w1_ref, b1_ref, o_ref = refs[1 + 2 * S:]
    s = s_refs[0][0] + s_refs[0][1]
    c = jnp.sum(c_refs[0][...], axis=0)
    for i in range(1, S):
        s = s + s_refs[i][0] + s_refs[i][1]
        c = c + jnp.sum(c_refs[i][...], axis=0)
    agg = s / jnp.maximum(c, 1.0)[:, None]
    h = _gelu(
        jnp.dot(x_ref[...], w0a_ref[...], preferred_element_type=jnp.float32)
        + jnp.dot(agg, w0b_ref[...], preferred_element_type=jnp.float32)
        + b0_ref[...]
    )
    o_ref[...] = _gelu(
        jnp.dot(h, w1_ref[...], preferred_element_type=jnp.float32) + b1_ref[...]
    )


def _node_ffn(nodes_pad, sums_list, counts_list, w0a, w0b, b0, w1, b1):
    nblk = N_PAD // BN
    return pl.pallas_call(
        _node_ffn_body,
        grid=(nblk,),
        in_specs=[pl.BlockSpec((BN, D), lambda i: (i, 0))]
        + [pl.BlockSpec((NCORES, BN, D), lambda i: (0, i, 0)) for _ in range(S)]
        + [pl.BlockSpec((NW, BN), lambda i: (0, i)) for _ in range(S)]
        + [
            pl.BlockSpec((D, D), lambda i: (0, 0)),
            pl.BlockSpec((D, D), lambda i: (0, 0)),
            pl.BlockSpec((1, D), lambda i: (0, 0)),
            pl.BlockSpec((D, D), lambda i: (0, 0)),
            pl.BlockSpec((1, D), lambda i: (0, 0)),
        ],
        out_specs=pl.BlockSpec((BN, D), lambda i: (i, 0)),
        out_shape=jax.ShapeDtypeStruct((N_PAD, D), jnp.float32),
    )(nodes_pad, *sums_list, *counts_list, w0a, w0b, b0, w1, b1)


def kernel(node_repesentations, edges, edge_features, Wp0, bp0, Wp1, bp1, Wu0, bu0, Wu1, bu1):
    e = edges.shape[1]
    nodes_pad = jnp.pad(node_repesentations, ((0, N_PAD - N), (0, 0)))
    src = jnp.pad(edges[0].astype(jnp.int32), (0, E_PAD - e)).reshape(E_PAD // CH, CH)
    dst = jnp.pad(
        edges[1].astype(jnp.int32), (0, E_PAD - e), constant_values=N_PAD - 1
    ).reshape(E_PAD // CH, CH)
    efT = edge_features.T
    full = (e // E_S) * E_S
    efT_tail = jnp.pad(efT[:, full:], ((0, 0), (0, S * E_S - e)))

    w0n = Wp0[DE:]
    w0e = Wp0[:DE]
    b1 = bp1.reshape(1, D)
    rows = E_S // CH
    g_list = []
    counts_list = []
    for s in range(S):
        src_s = lax.slice_in_dim(src, s * rows, (s + 1) * rows, axis=0)
        g_s, counts_s = _sc_gather(nodes_pad, src_s,
                                   lax.slice_in_dim(dst, s * rows, (s + 1) * rows, axis=0))
        g_list.append(g_s)
        counts_list.append(counts_s)

    b0 = bp0.reshape(1, D)

    sums_list = []
    for s in range(S):
        dst_s = lax.slice_in_dim(dst, s * rows, (s + 1) * rows, axis=0)
        if (s + 1) * E_S <= full:
            msgs_s = _edge_ffn(g_list[s], efT, s * E_S, w0n, w0e, b0, Wp1, b1)
        else:
            msgs_s = _edge_ffn(g_list[s], efT_tail, s * E_S - full, w0n, w0e, b0, Wp1, b1)
        sums_list.append(_sc_scatter(msgs_s, dst_s))

    out = _node_ffn(
        nodes_pad, sums_list, counts_list,
        Wu0[:D], Wu0[D:], bu0.reshape(1, D), Wu1, bu1.reshape(1, D),
    )
    return out[:N]


# speedup vs baseline: 1.0244x; 1.0244x over previous
import functools

import jax
import jax.numpy as jnp
from jax import lax
from jax.experimental import pallas as pl
from jax.experimental.pallas import tpu as pltpu
from jax.experimental.pallas import tpu_sc as plsc

N = 10000
D = 128
DE = 16

NCORES = 2
NSUB = 16
NW = NCORES * NSUB
CH = 128
NB = 4
NBS = 2

N_PAD = 10240
E_PAD = 327680
S = 5
E_S = E_PAD // S
RPW = E_S // NW
NCH = RPW // CH
NGROUP = NCH // NB
RPS = N_PAD // NSUB
BE = 2048
BN = 2048

_SC_PARAMS = pltpu.CompilerParams(needs_layout_passes=False)


def _gelu(x):
    return jax.nn.gelu(x)


def _sc_gather(table, idx2d, dst2d):
    mesh = plsc.VectorSubcoreMesh(core_axis_name="c", subcore_axis_name="s")

    @functools.partial(
        pl.kernel,
        out_type=(
            jax.ShapeDtypeStruct((E_S, D), jnp.float32),
            jax.ShapeDtypeStruct((NW, N_PAD), jnp.float32),
        ),
        mesh=mesh,
        compiler_params=_SC_PARAMS,
        scratch_types=[
            pltpu.VMEM((NCH, CH), jnp.int32),
            pltpu.VMEM((NCH, CH), jnp.int32),
            pltpu.VMEM((N_PAD,), jnp.float32),
        ]
        + [pltpu.VMEM((CH, D), jnp.float32) for _ in range(NB)]
        + [pltpu.SemaphoreType.DMA for _ in range(2 * NB)],
    )
    def k(table_hbm, idx_hbm, dst_hbm, out_hbm, counts_hbm, idx_all, dst_all, cnt_v, *rest):
        bufs = rest[:NB]
        gs = rest[NB:2 * NB]
        ws = rest[2 * NB:]
        wid = lax.axis_index("s") * NCORES + lax.axis_index("c")
        base = wid * RPW
        pltpu.sync_copy(idx_hbm.at[pl.ds(wid * NCH, NCH)], idx_all)
        pltpu.sync_copy(dst_hbm.at[pl.ds(wid * NCH, NCH)], dst_all)
        z16 = jnp.zeros((16,), jnp.float32)
        ones16 = jnp.ones((16,), jnp.float32)

        def zcnt(i, carry):
            cnt_v[pl.ds(i * 16, 16)] = z16
            return carry

        lax.fori_loop(0, N_PAD // 16, zcnt, 0)

        def count(j, carry):
            for kk in range(CH // 16):
                iv = dst_all[j, pl.ds(kk * 16, 16)]
                plsc.addupdate_scatter(cnt_v, [iv], ones16)
            return carry

        def g_start(j, b):
            pltpu.make_async_copy(table_hbm.at[idx_all.at[j]], bufs[b], gs[b]).start()

        def g_wait(j, b):
            pltpu.make_async_copy(table_hbm.at[idx_all.at[j]], bufs[b], gs[b]).wait()

        def w_start(j, b):
            pltpu.make_async_copy(
                bufs[b], out_hbm.at[pl.ds(base + j * CH, CH), :], ws[b]
            ).start()

        def w_wait(j, b):
            pltpu.make_async_copy(
                bufs[b], out_hbm.at[pl.ds(base + j * CH, CH), :], ws[b]
            ).wait()

        LA = NB - 2

        def step(j, b, start_next, drain_prev):
            bn = (b + LA) % NB
            if start_next:
                if drain_prev:
                    w_wait(j + LA - NB, bn)
                g_start(j + LA, bn)
            g_wait(j, b)
            w_start(j, b)

        for j in range(LA):
            g_start(j, j)
        for j in range(NB):
            step(j, j, True, j + LA >= NB)

        def body(g, carry):
            for b in range(NB):
                step(g * NB + b, b, True, True)
            return carry

        lax.fori_loop(1, NGROUP - 1, body, 0)

        for b in range(NB):
            j = (NGROUP - 1) * NB + b
            step(j, b, j + LA < NCH, True)

        lax.fori_loop(0, NCH, count, 0)
        pltpu.sync_copy(cnt_v, counts_hbm.at[wid])
        for b in range(NB):
            w_wait(NCH - NB + b, (NCH - NB + b) % NB)

    return k(table, idx2d, dst2d)


def _edge_ffn_body(g_ref, ef_ref, w0n_ref, w0e_ref, b0_ref, w1_ref, b1_ref, o_ref):
    q = lax.dot_general(
        ef_ref[...], w0e_ref[...], (((0,), (0,)), ((), ())),
        preferred_element_type=jnp.float32,
    )
    pre = (
        jnp.dot(g_ref[...], w0n_ref[...], preferred_element_type=jnp.float32)
        + q
        + b0_ref[...]
    )
    h = _gelu(pre)
    o_ref[...] = _gelu(
        jnp.dot(h, w1_ref[...], preferred_element_type=jnp.float32) + b1_ref[...]
    )


def _edge_ffn(g, efT, ef_col0, w0n, w0e, b0, w1, b1):
    nblk = E_S // BE
    blk0 = ef_col0 // BE
    return pl.pallas_call(
        _edge_ffn_body,
        grid=(nblk,),
        in_specs=[
            pl.BlockSpec((BE, D), lambda i: (i, 0)),
            pl.BlockSpec((DE, BE), lambda i: (0, blk0 + i)),
            pl.BlockSpec((D, D), lambda i: (0, 0)),
            pl.BlockSpec((DE, D), lambda i: (0, 0)),
            pl.BlockSpec((1, D), lambda i: (0, 0)),
            pl.BlockSpec((D, D), lambda i: (0, 0)),
            pl.BlockSpec((1, D), lambda i: (0, 0)),
        ],
        out_specs=pl.BlockSpec((BE, D), lambda i: (i, 0)),
        out_shape=jax.ShapeDtypeStruct((E_S, D), jnp.float32),
    )(g, efT, w0n, w0e, b0, w1, b1)


def _sc_scatter(msgs, dst2d):
    mesh = plsc.VectorSubcoreMesh(core_axis_name="c", subcore_axis_name="s")
    ngroup = NCH // NBS

    @functools.partial(
        pl.kernel,
        out_type=jax.ShapeDtypeStruct((NCORES, N_PAD, D), jnp.float32),
        mesh=mesh,
        compiler_params=_SC_PARAMS,
        scratch_types=[pltpu.VMEM((NCH, CH), jnp.int32)]
        + [pltpu.VMEM((CH, D), jnp.float32) for _ in range(NBS)]
        + [pltpu.VMEM_SHARED((N_PAD, D), jnp.float32)]
        + [pltpu.SemaphoreType.DMA for _ in range(NBS)],
    )
    def k(msgs_hbm, dst_hbm, sums_hbm, idx_all, *rest):
        ms = rest[:NBS]
        acc_sh = rest[NBS]
        ss = rest[NBS + 1:]
        cid = lax.axis_index("c")
        sid = lax.axis_index("s")
        wid = sid * NCORES + cid
        z16 = jnp.zeros((16,), jnp.float32)

        def zbuf(i, carry):
            r = i // (D // 16)
            c = (i % (D // 16)) * 16
            ms[0][r, pl.ds(c, 16)] = z16
            return carry

        lax.fori_loop(0, CH * (D // 16), zbuf, 0)

        for j in range(RPS // CH):
            pltpu.sync_copy(ms[0], acc_sh.at[pl.ds(sid * RPS + j * CH, CH), :])
        plsc.subcore_barrier()

        pltpu.sync_copy(dst_hbm.at[pl.ds(wid * NCH, NCH)], idx_all)
        base = wid * RPW

        def l_start(j, b):
            pltpu.make_async_copy(
                msgs_hbm.at[pl.ds(base + j * CH, CH), :], ms[b], ss[b]
            ).start()

        def l_wait(j, b):
            pltpu.make_async_copy(
                msgs_hbm.at[pl.ds(base + j * CH, CH), :], ms[b], ss[b]
            ).wait()

        def consume(j, b):
            pltpu.sync_copy(ms[b], acc_sh.at[idx_all.at[j]], add=True)

        for b in range(NBS):
            l_start(b, b)

        def group(g, carry):
            for b in range(NBS):
                j = g * NBS + b
                l_wait(j, b)
                consume(j, b)
                l_start((g + 1) * NBS + b, b)
            return carry

        lax.fori_loop(0, ngroup - 1, group, 0)

        gl = ngroup - 1
        for b in range(NBS):
            j = gl * NBS + b
            l_wait(j, b)
            consume(j, b)
        plsc.subcore_barrier()

        for j in range(RPS // CH):
            r0 = sid * RPS + j * CH
            pltpu.sync_copy(acc_sh.at[pl.ds(r0, CH), :], ms[0])
            pltpu.sync_copy(ms[0], sums_hbm.at[cid, pl.ds(r0, CH), :])

    return k(msgs, dst2d)


def _node_ffn_body(*refs):
    x_ref = refs[0]
    s_refs = refs[1:1 + S]
    c_refs = refs[1 + S:1 + 2 * S]
    w0a_ref, w0b_ref, b0_ref, w1_ref, b1_ref, o_ref = refs[1 + 2 * S:]
    s = s_refs[0][0] + s_refs[0][1]
    c = jnp.sum(c_refs[0][...], axis=0)
    for i in range(1, S):
        s = s + s_refs[i][0] + s_refs[i][1]
        c = c + jnp.sum(c_refs[i][...], axis=0)
    agg = s / jnp.maximum(c, 1.0)[:, None]
    h = _gelu(
        jnp.dot(x_ref[...], w0a_ref[...], preferred_element_type=jnp.float32)
        + jnp.dot(agg, w0b_ref[...], preferred_element_type=jnp.float32)
        + b0_ref[...]
    )
    o_ref[...] = _gelu(
        jnp.dot(h, w1_ref[...], preferred_element_type=jnp.float32) + b1_ref[...]
    )


def _node_ffn(nodes_pad, sums_list, counts_list, w0a, w0b, b0, w1, b1):
    nblk = N_PAD // BN
    return pl.pallas_call(
        _node_ffn_body,
        grid=(nblk,),
        in_specs=[pl.BlockSpec((BN, D), lambda i: (i, 0))]
        + [pl.BlockSpec((NCORES, BN, D), lambda i: (0, i, 0)) for _ in range(S)]
        + [pl.BlockSpec((NW, BN), lambda i: (0, i)) for _ in range(S)]
        + [
            pl.BlockSpec((D, D), lambda i: (0, 0)),
            pl.BlockSpec((D, D), lambda i: (0, 0)),
            pl.BlockSpec((1, D), lambda i: (0, 0)),
            pl.BlockSpec((D, D), lambda i: (0, 0)),
            pl.BlockSpec((1, D), lambda i: (0, 0)),
        ],
        out_specs=pl.BlockSpec((BN, D), lambda i: (i, 0)),
        out_shape=jax.ShapeDtypeStruct((N_PAD, D), jnp.float32),
    )(nodes_pad, *sums_list, *counts_list, w0a, w0b, b0, w1, b1)


def kernel(node_repesentations, edges, edge_features, Wp0, bp0, Wp1, bp1, Wu0, bu0, Wu1, bu1):
    e = edges.shape[1]
    nodes_pad = jnp.pad(node_repesentations, ((0, N_PAD - N), (0, 0)))
    src = jnp.pad(edges[0].astype(jnp.int32), (0, E_PAD - e)).reshape(E_PAD // CH, CH)
    dst = jnp.pad(
        edges[1].astype(jnp.int32), (0, E_PAD - e), constant_values=N_PAD - 1
    ).reshape(E_PAD // CH, CH)
    efT = edge_features.T
    full = (e // E_S) * E_S
    efT_tail = jnp.pad(efT[:, full:], ((0, 0), (0, S * E_S - e)))

    w0n = Wp0[DE:]
    w0e = Wp0[:DE]
    b1 = bp1.reshape(1, D)
    rows = E_S // CH
    g_list = []
    counts_list = []
    for s in range(S):
        src_s = lax.slice_in_dim(src, s * rows, (s + 1) * rows, axis=0)
        g_s, counts_s = _sc_gather(node_repesentations, src_s,
                                   lax.slice_in_dim(dst, s * rows, (s + 1) * rows, axis=0))
        g_list.append(g_s)
        counts_list.append(counts_s)

    b0 = bp0.reshape(1, D)

    sums_list = []
    for s in range(S):
        dst_s = lax.slice_in_dim(dst, s * rows, (s + 1) * rows, axis=0)
        if (s + 1) * E_S <= full:
            msgs_s = _edge_ffn(g_list[s], efT, s * E_S, w0n, w0e, b0, Wp1, b1)
        else:
            msgs_s = _edge_ffn(g_list[s], efT_tail, s * E_S - full, w0n, w0e, b0, Wp1, b1)
        sums_list.append(_sc_scatter(msgs_s, dst_s))

    out = _node_ffn(
        nodes_pad, sums_list, counts_list,
        Wu0[:D], Wu0[D:], bu0.reshape(1, D), Wu1, bu1.reshape(1, D),
    )
    return out[:N]
